# baseline (speedup 1.0000x reference)
import jax
import jax.numpy as jnp
from jax import lax
from jax.experimental import pallas as pl
from jax.experimental.pallas import tpu as pltpu
from jax.experimental.pallas import tpu_sc as plsc

N_FIELDS = 26
VOCAB = 100000
D_TOKEN = 64
BATCH = 16384

NC = 2
NS = 16
L = 16
NW = NC * NS
N_ROWS = N_FIELDS * D_TOKEN
ROWS_PER_W = N_ROWS // NW

TSPLIT = 50048
HB = BATCH // 2
SC_CHUNK = 4096
POS_BITS = 13
POS_MASK = HB - 1


def _body(xc_hbm, tab_hbm, out_hbm, bufA, bufB, combo, out_v, stage, ns_s,
          semA, semB):
    wid = lax.axis_index("s") * NC + lax.axis_index("c")
    r0 = wid * ROWS_PER_W
    iota16 = lax.iota(jnp.int32, L)

    def issue_tab(rr):
        fr = rr >> 6
        dr = rr & (D_TOKEN - 1)
        pltpu.async_copy(tab_hbm.at[fr, dr, pl.ds(0, TSPLIT)], bufA, semA)
        pltpu.async_copy(tab_hbm.at[fr, dr, pl.ds(TSPLIT, VOCAB - TSPLIT)],
                         bufB, semB)

    issue_tab(r0)

    def row_step(i, last_f):
        r = r0 + i
        f = r >> 6
        d = r & (D_TOKEN - 1)

        @pl.when(f != last_f)
        def _():
            for h in range(2):
                base = HB * h

                def cchunk(q, ptrs, base=base):
                    off = pl.multiple_of(base + SC_CHUNK * q, 128)
                    pltpu.sync_copy(
                        xc_hbm.at[f, pl.ds(off, SC_CHUNK)],
                        stage)

                    def citer(t, ptrs2, q=q):
                        pA, pB = ptrs2
                        w = stage[pl.ds(t * L, L)]
                        pos = t * L + SC_CHUNK * q + iota16
                        mA = w < TSPLIT
                        miA = mA.astype(jnp.int32)
                        csA = plsc.cumsum(miA)
                        wsh = w << POS_BITS
                        plsc.store_scatter(combo, [pA - 1 + csA], wsh | pos,
                                           mask=mA)
                        totA = jnp.sum(miA)
                        csB = plsc.cumsum(1 - miA)
                        plsc.store_scatter(
                            combo, [pB - csB],
                            (wsh - (TSPLIT << POS_BITS)) | pos,
                            mask=jnp.logical_not(mA))
                        return (pA + totA, pB - (L - totA))

                    return lax.fori_loop(0, SC_CHUNK // L, citer, ptrs)

                ptrA_f, ptrB_f = lax.fori_loop(
                    0, HB // SC_CHUNK, cchunk,
                    (jnp.int32(base), jnp.int32(base + HB)))
                ns_s[2 * h] = ptrA_f - base
                ns_s[2 * h + 1] = base + HB - ptrB_f

        pltpu.make_async_copy(
            tab_hbm.at[f, d, pl.ds(0, TSPLIT)], bufA, semA).wait()
        pltpu.make_async_copy(
            tab_hbm.at[f, d, pl.ds(TSPLIT, VOCAB - TSPLIT)], bufB,
            semB).wait()

        for h in range(2):
            base = HB * h
            nA = ns_s[2 * h]

            def blk(buf, off):
                w = combo[pl.ds(off, L)]
                vals = plsc.load_gather(buf, [w >> POS_BITS])
                plsc.store_scatter(out_v, [w & POS_MASK], vals)

            kfull = nA >> 4
            k4 = kfull >> 2

            def a4(j, c2, base=base):
                for u in range(4):
                    blk(bufA, base + j * 64 + u * L)
                return c2

            def a1(j, c2, base=base):
                blk(bufA, base + j * L)
                return c2

            lax.fori_loop(0, k4, a4, 0)
            lax.fori_loop(k4 * 4, kfull, a1, 0)

            bnd = base + (nA & ~(L - 1))
            has_bnd = (nA & (L - 1)) > 0

            @pl.when(has_bnd)
            def _(base=base, bnd=bnd, nA=nA):
                w = combo[pl.ds(bnd, L)]
                msk = (bnd + iota16) < base + nA
                vals = plsc.load_gather(bufA, [w >> POS_BITS], mask=msk)
                plsc.store_scatter(out_v, [w & POS_MASK], vals, mask=msk)

            if h == 1:
                @pl.when(i + 1 < ROWS_PER_W)
                def _():
                    rn = r + 1
                    pltpu.async_copy(
                        tab_hbm.at[rn >> 6, rn & (D_TOKEN - 1),
                                   pl.ds(0, TSPLIT)], bufA, semA)

            @pl.when(has_bnd)
            def _(base=base, bnd=bnd, nA=nA):
                w = combo[pl.ds(bnd, L)]
                msk = (bnd + iota16) >= base + nA
                vals = plsc.load_gather(bufB, [w >> POS_BITS], mask=msk)
                plsc.store_scatter(out_v, [w & POS_MASK], vals, mask=msk)

            sB = (nA + L - 1) >> 4
            kB = (HB >> 4) - sB
            kB4 = kB >> 2

            def b4(j, c2, base=base, sB=sB):
                for u in range(4):
                    blk(bufB, base + (sB + j * 4 + u) * L)
                return c2

            def b1(j, c2, base=base, sB=sB):
                blk(bufB, base + (sB + j) * L)
                return c2

            lax.fori_loop(0, kB4, b4, 0)
            lax.fori_loop(kB4 * 4, kB, b1, 0)

            if h == 1:
                @pl.when(i + 1 < ROWS_PER_W)
                def _():
                    rn = r + 1
                    pltpu.async_copy(
                        tab_hbm.at[rn >> 6, rn & (D_TOKEN - 1),
                                   pl.ds(TSPLIT, VOCAB - TSPLIT)], bufB,
                        semB)

            pltpu.sync_copy(out_v, out_hbm.at[f, d, pl.ds(base, HB)])

        return f

    lax.fori_loop(0, ROWS_PER_W, row_step, -1)


@jax.jit
def _lookup(xc_t, tab_t):
    mesh = plsc.VectorSubcoreMesh(core_axis_name="c", subcore_axis_name="s")
    return pl.kernel(
        _body,
        mesh=mesh,
        out_type=jax.ShapeDtypeStruct((N_FIELDS, D_TOKEN, BATCH),
                                      jnp.float32),
        scratch_types=[
            pltpu.VMEM((TSPLIT,), jnp.float32),
            pltpu.VMEM((VOCAB - TSPLIT,), jnp.float32),
            pltpu.VMEM((BATCH,), jnp.int32),
            pltpu.VMEM((HB,), jnp.float32),
            pltpu.VMEM((SC_CHUNK,), jnp.int32),
            pltpu.SMEM((8,), jnp.int32),
            pltpu.SemaphoreType.DMA,
            pltpu.SemaphoreType.DMA,
        ],
        compiler_params=pltpu.CompilerParams(needs_layout_passes=False),
    )(xc_t, tab_t)


def kernel(x_cat, tables):
    xc_t = x_cat.astype(jnp.int32).T
    tab_t = jnp.transpose(tables, (0, 2, 1))
    out_t = _lookup(xc_t, tab_t)
    return jnp.transpose(out_t, (2, 0, 1))
